# baseline (device time: 185544 ns/iter reference)
import jax
import jax.numpy as jnp
from jax import lax
from jax.experimental import pallas as pl
from jax.experimental.pallas import tpu as pltpu

N_DEV = 8
E_PER = 4


def kernel(x, router_W, route_idx, expert_W):
    n_tok, d = x.shape
    e_per, _, h = expert_W.shape
    n_hops = N_DEV - 1

    def body(x_ref, rw_ref, idx_ref, ew_ref, out_ref, comm_ref, send_sems, recv_sems):
        my = lax.axis_index("i")
        left = lax.rem(my + N_DEV - 1, N_DEV)
        right = lax.rem(my + 1, N_DEV)

        barrier_sem = pltpu.get_barrier_semaphore()
        for nbr in (left, right):
            pl.semaphore_signal(
                barrier_sem, inc=1,
                device_id=(nbr,), device_id_type=pltpu.DeviceIdType.MESH,
            )
        pl.semaphore_wait(barrier_sem, 2)

        xv = x_ref[:, :]
        scores = jnp.dot(xv, rw_ref[:, :], preferred_element_type=jnp.float32)
        s_max = jnp.max(scores, axis=-1, keepdims=True)
        p = jnp.exp(scores - s_max)
        probs = p / jnp.sum(p, axis=-1, keepdims=True)
        e0 = idx_ref[:, 0:1]
        e1 = idx_ref[:, 1:2]
        eids = lax.broadcasted_iota(jnp.int32, scores.shape, 1)
        g0 = jnp.sum(jnp.where(eids == e0, probs, 0.0), axis=-1, keepdims=True)
        g1 = jnp.sum(jnp.where(eids == e1, probs, 0.0), axis=-1, keepdims=True)
        gs = g0 + g1
        g0 = g0 / gs
        g1 = g1 / gs

        def accum(block_ref, owner, first):
            acc = jnp.zeros((n_tok, h), jnp.float32) if first else out_ref[:, :]
            for t in range(E_PER):
                e_id = owner * E_PER + t
                w = jnp.where(e0 == e_id, g0, 0.0) + jnp.where(e1 == e_id, g1, 0.0)
                acc = acc + jnp.dot(
                    xv * w, block_ref[t], preferred_element_type=jnp.float32
                )
            out_ref[:, :] = acc

        accum(ew_ref, my, True)

        for k in range(n_hops):
            src = ew_ref if k == 0 else comm_ref.at[k - 1]
            rdma = pltpu.make_async_remote_copy(
                src_ref=src,
                dst_ref=comm_ref.at[k],
                send_sem=send_sems.at[k],
                recv_sem=recv_sems.at[k],
                device_id=(right,),
                device_id_type=pltpu.DeviceIdType.MESH,
            )
            rdma.start()
            rdma.wait()
            owner = lax.rem(my - k - 1 + N_DEV, N_DEV)
            accum(comm_ref.at[k], owner, False)

    return pl.pallas_call(
        body,
        out_shape=jax.ShapeDtypeStruct((n_tok, h), jnp.float32),
        in_specs=[pl.BlockSpec(memory_space=pltpu.VMEM)] * 4,
        out_specs=pl.BlockSpec(memory_space=pltpu.VMEM),
        scratch_shapes=[
            pltpu.VMEM((n_hops, e_per, d, h), jnp.float32),
            pltpu.SemaphoreType.DMA((n_hops,)),
            pltpu.SemaphoreType.DMA((n_hops,)),
        ],
        compiler_params=pltpu.CompilerParams(collective_id=0),
    )(x, router_W, route_idx, expert_W)


# device time: 72005 ns/iter; 2.5768x vs baseline; 2.5768x over previous
import jax
import jax.numpy as jnp
from jax import lax
from jax.experimental import pallas as pl
from jax.experimental.pallas import tpu as pltpu

N_DEV = 8
E_PER = 4


def kernel(x, router_W, route_idx, expert_W):
    n_tok, d = x.shape
    e_per, _, h = expert_W.shape

    def body(x_ref, rw_ref, idx_ref, ew_ref, out_ref, all_ref, send_sems, recv_sems):
        my = lax.axis_index("i")

        q = lax.rem(my, 4)
        cz = my // 4
        cy = q // 2
        cx = lax.rem(q + cy, 2)

        def pos(px, py, pz):
            return pz * 4 + 2 * py + lax.rem(px + py, 2)

        nx = pos(1 - cx, cy, cz)
        ny = pos(cx, 1 - cy, cz)
        nz = pos(cx, cy, 1 - cz)
        fxy = pos(1 - cx, 1 - cy, cz)
        fxz = pos(1 - cx, cy, 1 - cz)
        fyz = pos(cx, 1 - cy, 1 - cz)
        fxyz = pos(1 - cx, 1 - cy, 1 - cz)

        barrier_sem = pltpu.get_barrier_semaphore()
        for nbr in (nx, ny, nz):
            pl.semaphore_signal(
                barrier_sem, inc=1,
                device_id=(nbr,), device_id_type=pltpu.DeviceIdType.MESH,
            )
        pl.semaphore_wait(barrier_sem, 3)

        xv = x_ref[:, :]
        scores = jnp.dot(xv, rw_ref[:, :], preferred_element_type=jnp.float32)
        s_max = jnp.max(scores, axis=-1, keepdims=True)
        p = jnp.exp(scores - s_max)
        probs = p / jnp.sum(p, axis=-1, keepdims=True)
        e0 = idx_ref[:, 0:1]
        e1 = idx_ref[:, 1:2]
        eids = lax.broadcasted_iota(jnp.int32, scores.shape, 1)
        g0 = jnp.sum(jnp.where(eids == e0, probs, 0.0), axis=-1, keepdims=True)
        g1 = jnp.sum(jnp.where(eids == e1, probs, 0.0), axis=-1, keepdims=True)
        gs = g0 + g1
        g0 = g0 / gs
        g1 = g1 / gs

        def accum(block_ref, owner, first=False):
            ws = []
            for t in range(E_PER):
                e_id = owner * E_PER + t
                w = jnp.where(e0 == e_id, g0, 0.0) + jnp.where(e1 == e_id, g1, 0.0)
                ws.append(xv * w)
            xcat = jnp.concatenate(ws, axis=1)
            wmat = block_ref[...].reshape(E_PER * d, h)
            contrib = jnp.dot(xcat, wmat, preferred_element_type=jnp.float32)
            out_ref[:, :] = contrib if first else out_ref[:, :] + contrib

        def copy(src, dst, dev, phase, link):
            return pltpu.make_async_remote_copy(
                src_ref=src, dst_ref=dst,
                send_sem=send_sems.at[phase, link],
                recv_sem=recv_sems.at[phase, link],
                device_id=(dev,), device_id_type=pltpu.DeviceIdType.MESH,
            )

        def run_phase(rdmas, accums):
            for r in rdmas:
                r.start()
            for blk, owner in accums:
                accum(blk, owner)
            for r in rdmas:
                r.wait_send()
            for r in rdmas:
                r.wait_recv()

        p1 = [
            copy(ew_ref, all_ref.at[my], nx, 0, 0),
            copy(ew_ref, all_ref.at[my], ny, 0, 1),
            copy(ew_ref, all_ref.at[my], nz, 0, 2),
        ]
        for r in p1:
            r.start()
        accum(ew_ref, my, first=True)
        for r in p1:
            r.wait_send()
        for r in p1:
            r.wait_recv()

        run_phase(
            [
                copy(all_ref.at[ny], all_ref.at[ny], nx, 1, 0),
                copy(all_ref.at[nz], all_ref.at[nz], ny, 1, 1),
                copy(all_ref.at[nx], all_ref.at[nx], nz, 1, 2),
            ],
            [(all_ref.at[nx], nx), (all_ref.at[ny], ny), (all_ref.at[nz], nz)],
        )

        run_phase(
            [
                copy(all_ref.at[fyz, 0:2], all_ref.at[fyz, 0:2], nx, 2, 0),
                copy(all_ref.at[fxz, 2:3], all_ref.at[fxz, 2:3], ny, 2, 1),
                copy(all_ref.at[fxy, 3:4], all_ref.at[fxy, 3:4], nz, 2, 2),
            ],
            [(all_ref.at[fxy], fxy), (all_ref.at[fyz], fyz), (all_ref.at[fxz], fxz)],
        )

        accum(all_ref.at[fxyz], fxyz)

    return pl.pallas_call(
        body,
        out_shape=jax.ShapeDtypeStruct((n_tok, h), jnp.float32),
        in_specs=[pl.BlockSpec(memory_space=pltpu.VMEM)] * 4,
        out_specs=pl.BlockSpec(memory_space=pltpu.VMEM),
        scratch_shapes=[
            pltpu.VMEM((N_DEV, e_per, d, h), jnp.float32),
            pltpu.SemaphoreType.DMA((3, 3)),
            pltpu.SemaphoreType.DMA((3, 3)),
        ],
        compiler_params=pltpu.CompilerParams(collective_id=0),
    )(x, router_W, route_idx, expert_W)


# device time: 44324 ns/iter; 4.1861x vs baseline; 1.6245x over previous
import jax
import jax.numpy as jnp
from jax import lax
from jax.experimental import pallas as pl
from jax.experimental.pallas import tpu as pltpu

N_DEV = 8
E_PER = 4


def kernel(x, router_W, route_idx, expert_W):
    n_tok, d = x.shape
    e_per, _, h = expert_W.shape

    def body(x_ref, rw_ref, idx_ref, ew_ref, out_ref, all_ref, my_bf, send_sems, recv_sems):
        my = lax.axis_index("i")

        q = lax.rem(my, 4)
        cz = my // 4
        cy = q // 2
        cx = lax.rem(q + cy, 2)

        def pos(px, py, pz):
            return pz * 4 + 2 * py + lax.rem(px + py, 2)

        nx = pos(1 - cx, cy, cz)
        ny = pos(cx, 1 - cy, cz)
        nz = pos(cx, cy, 1 - cz)
        fxy = pos(1 - cx, 1 - cy, cz)
        fxz = pos(1 - cx, cy, 1 - cz)
        fyz = pos(cx, 1 - cy, 1 - cz)
        fxyz = pos(1 - cx, 1 - cy, 1 - cz)

        barrier_sem = pltpu.get_barrier_semaphore()
        for nbr in (nx, ny, nz):
            pl.semaphore_signal(
                barrier_sem, inc=1,
                device_id=(nbr,), device_id_type=pltpu.DeviceIdType.MESH,
            )
        pl.semaphore_wait(barrier_sem, 3)

        xv = x_ref[:, :]
        scores = jnp.dot(xv, rw_ref[:, :], preferred_element_type=jnp.float32)
        s_max = jnp.max(scores, axis=-1, keepdims=True)
        p = jnp.exp(scores - s_max)
        probs = p / jnp.sum(p, axis=-1, keepdims=True)
        e0 = idx_ref[:, 0:1]
        e1 = idx_ref[:, 1:2]
        eids = lax.broadcasted_iota(jnp.int32, scores.shape, 1)
        g0 = jnp.sum(jnp.where(eids == e0, probs, 0.0), axis=-1, keepdims=True)
        g1 = jnp.sum(jnp.where(eids == e1, probs, 0.0), axis=-1, keepdims=True)
        gs = g0 + g1
        g0 = g0 / gs
        g1 = g1 / gs

        def accum(block_ref, owner, first=False):
            ws = []
            for t in range(E_PER):
                e_id = owner * E_PER + t
                w = jnp.where(e0 == e_id, g0, 0.0) + jnp.where(e1 == e_id, g1, 0.0)
                ws.append(xv * w)
            xcat = jnp.concatenate(ws, axis=1)
            wmat = block_ref[...].reshape(E_PER * d, h)
            if wmat.dtype != jnp.float32:
                xcat = xcat.astype(wmat.dtype)
            contrib = jnp.dot(xcat, wmat, preferred_element_type=jnp.float32)
            out_ref[:, :] = contrib if first else out_ref[:, :] + contrib

        def copy(src, dst, dev, phase, link):
            return pltpu.make_async_remote_copy(
                src_ref=src, dst_ref=dst,
                send_sem=send_sems.at[phase, link],
                recv_sem=recv_sems.at[phase, link],
                device_id=(dev,), device_id_type=pltpu.DeviceIdType.MESH,
            )

        def run_phase(rdmas, accums):
            for r in rdmas:
                r.start()
            for blk, owner in accums:
                accum(blk, owner)
            for r in rdmas:
                r.wait_send()
            for r in rdmas:
                r.wait_recv()

        my_bf[...] = ew_ref[...].astype(jnp.bfloat16)

        p1 = [
            copy(my_bf, all_ref.at[my], nx, 0, 0),
            copy(my_bf, all_ref.at[my], ny, 0, 1),
            copy(my_bf, all_ref.at[my], nz, 0, 2),
        ]
        for r in p1:
            r.start()
        accum(ew_ref, my, first=True)
        for r in p1:
            r.wait_send()
        for r in p1:
            r.wait_recv()

        run_phase(
            [
                copy(all_ref.at[ny], all_ref.at[ny], nx, 1, 0),
                copy(all_ref.at[nz], all_ref.at[nz], ny, 1, 1),
                copy(all_ref.at[nx], all_ref.at[nx], nz, 1, 2),
            ],
            [(all_ref.at[nx], nx), (all_ref.at[ny], ny), (all_ref.at[nz], nz)],
        )

        run_phase(
            [
                copy(all_ref.at[fyz, 0:2], all_ref.at[fyz, 0:2], nx, 2, 0),
                copy(all_ref.at[fxz, 2:3], all_ref.at[fxz, 2:3], ny, 2, 1),
                copy(all_ref.at[fxy, 3:4], all_ref.at[fxy, 3:4], nz, 2, 2),
            ],
            [(all_ref.at[fxy], fxy), (all_ref.at[fyz], fyz), (all_ref.at[fxz], fxz)],
        )

        accum(all_ref.at[fxyz], fxyz)

    return pl.pallas_call(
        body,
        out_shape=jax.ShapeDtypeStruct((n_tok, h), jnp.float32),
        in_specs=[pl.BlockSpec(memory_space=pltpu.VMEM)] * 4,
        out_specs=pl.BlockSpec(memory_space=pltpu.VMEM),
        scratch_shapes=[
            pltpu.VMEM((N_DEV, e_per, d, h), jnp.bfloat16),
            pltpu.VMEM((e_per, d, h), jnp.bfloat16),
            pltpu.SemaphoreType.DMA((3, 3)),
            pltpu.SemaphoreType.DMA((3, 3)),
        ],
        compiler_params=pltpu.CompilerParams(collective_id=0),
    )(x, router_W, route_idx, expert_W)


# device time: 40875 ns/iter; 4.5393x vs baseline; 1.0844x over previous
import jax
import jax.numpy as jnp
from jax import lax
from jax.experimental import pallas as pl
from jax.experimental.pallas import tpu as pltpu

N_DEV = 8
E_PER = 4
COMM_DTYPE = jnp.bfloat16


def kernel(x, router_W, route_idx, expert_W):
    n_tok, d = x.shape
    e_per, _, h = expert_W.shape

    def body(x_ref, rw_ref, idx_ref, ew_ref, out_ref, all_ref, my_q, send_sems, recv_sems):
        my = lax.axis_index("i")

        q = lax.rem(my, 4)
        cz = my // 4
        cy = q // 2
        cx = lax.rem(q + cy, 2)

        def pos(px, py, pz):
            return pz * 4 + 2 * py + lax.rem(px + py, 2)

        nx = pos(1 - cx, cy, cz)
        ny = pos(cx, 1 - cy, cz)
        nz = pos(cx, cy, 1 - cz)
        fxy = pos(1 - cx, 1 - cy, cz)
        fxz = pos(1 - cx, cy, 1 - cz)
        fyz = pos(cx, 1 - cy, 1 - cz)
        fxyz = pos(1 - cx, 1 - cy, 1 - cz)

        xv = x_ref[:, :]
        scores = jnp.dot(xv, rw_ref[:, :], preferred_element_type=jnp.float32)
        s_max = jnp.max(scores, axis=-1, keepdims=True)
        p = jnp.exp(scores - s_max)
        probs = p / jnp.sum(p, axis=-1, keepdims=True)
        e0 = idx_ref[:, 0:1]
        e1 = idx_ref[:, 1:2]
        eids = lax.broadcasted_iota(jnp.int32, scores.shape, 1)
        g0 = jnp.sum(jnp.where(eids == e0, probs, 0.0), axis=-1, keepdims=True)
        g1 = jnp.sum(jnp.where(eids == e1, probs, 0.0), axis=-1, keepdims=True)
        gs = g0 + g1
        g0 = g0 / gs
        g1 = g1 / gs

        my_q[...] = ew_ref[...].astype(COMM_DTYPE)

        def accum(block_ref, owner, first=False):
            ws = []
            for t in range(E_PER):
                e_id = owner * E_PER + t
                w = jnp.where(e0 == e_id, g0, 0.0) + jnp.where(e1 == e_id, g1, 0.0)
                ws.append(xv * w)
            xcat = jnp.concatenate(ws, axis=1)
            wmat = block_ref[...].reshape(E_PER * d, h)
            if wmat.dtype != jnp.float32:
                xcat = xcat.astype(wmat.dtype)
            contrib = jnp.dot(xcat, wmat, preferred_element_type=jnp.float32)
            out_ref[:, :] = contrib if first else out_ref[:, :] + contrib

        def copy(src, dst, dev, row, link):
            return pltpu.make_async_remote_copy(
                src_ref=src, dst_ref=dst,
                send_sem=send_sems.at[row, link],
                recv_sem=recv_sems.at[row, link],
                device_id=(dev,), device_id_type=pltpu.DeviceIdType.MESH,
            )

        barrier_sem = pltpu.get_barrier_semaphore()
        for nbr in (nx, ny, nz):
            pl.semaphore_signal(
                barrier_sem, inc=1,
                device_id=(nbr,), device_id_type=pltpu.DeviceIdType.MESH,
            )
        pl.semaphore_wait(barrier_sem, 3)

        HALF = [slice(0, 2), slice(2, 4)]
        devs = (nx, ny, nz)

        p1 = [
            [copy(my_q.at[HALF[hh]], all_ref.at[my, HALF[hh]], devs[l], hh, l)
             for hh in range(2)]
            for l in range(3)
        ]
        for l in range(3):
            for hh in range(2):
                p1[l][hh].start()

        accum(ew_ref, my, first=True)

        src_slot = (ny, nz, nx)
        p2 = [
            [copy(all_ref.at[src_slot[l], HALF[hh]],
                  all_ref.at[src_slot[l], HALF[hh]], devs[l], 2 + hh, l)
             for hh in range(2)]
            for l in range(3)
        ]
        unlock = [(1, 0), (2, 1), (0, 2)]
        for hh in range(2):
            for rl, sl in unlock:
                p1[rl][hh].wait_recv()
                p2[sl][hh].start()

        accum(all_ref.at[nx], nx)
        accum(all_ref.at[ny], ny)
        accum(all_ref.at[nz], nz)

        p3 = [
            copy(all_ref.at[fyz, 0:2], all_ref.at[fyz, 0:2], nx, 4, 0),
            copy(all_ref.at[fxz, 2:3], all_ref.at[fxz, 2:3], ny, 4, 1),
            copy(all_ref.at[fxy, 3:4], all_ref.at[fxy, 3:4], nz, 4, 2),
        ]
        p2[1][0].wait_recv()
        p3[0].start()
        p2[2][0].wait_recv()
        p2[2][1].wait_recv()
        p3[1].start()
        p2[0][0].wait_recv()
        p2[0][1].wait_recv()
        p3[2].start()
        p2[1][1].wait_recv()

        accum(all_ref.at[fxy], fxy)
        accum(all_ref.at[fyz], fyz)
        accum(all_ref.at[fxz], fxz)

        for r in p3:
            r.wait_recv()
        accum(all_ref.at[fxyz], fxyz)

        for l in range(3):
            for hh in range(2):
                p1[l][hh].wait_send()
                p2[l][hh].wait_send()
        for r in p3:
            r.wait_send()

    return pl.pallas_call(
        body,
        out_shape=jax.ShapeDtypeStruct((n_tok, h), jnp.float32),
        in_specs=[pl.BlockSpec(memory_space=pltpu.VMEM)] * 4,
        out_specs=pl.BlockSpec(memory_space=pltpu.VMEM),
        scratch_shapes=[
            pltpu.VMEM((N_DEV, e_per, d, h), COMM_DTYPE),
            pltpu.VMEM((e_per, d, h), COMM_DTYPE),
            pltpu.SemaphoreType.DMA((5, 3)),
            pltpu.SemaphoreType.DMA((5, 3)),
        ],
        compiler_params=pltpu.CompilerParams(collective_id=0),
    )(x, router_W, route_idx, expert_W)


# device time: 40342 ns/iter; 4.5993x vs baseline; 1.0132x over previous
import jax
import jax.numpy as jnp
from jax import lax
from jax.experimental import pallas as pl
from jax.experimental.pallas import tpu as pltpu

N_DEV = 8
E_PER = 4
COMM_DTYPE = jnp.bfloat16


def kernel(x, router_W, route_idx, expert_W):
    n_tok, d = x.shape
    e_per, _, h = expert_W.shape

    def body(x_ref, rw_ref, idx_ref, ew_ref, out_ref, all_ref, my_q, send_sems, recv_sems):
        my = lax.axis_index("i")

        q = lax.rem(my, 4)
        cz = my // 4
        cy = q // 2
        cx = lax.rem(q + cy, 2)

        def pos(px, py, pz):
            return pz * 4 + 2 * py + lax.rem(px + py, 2)

        nx = pos(1 - cx, cy, cz)
        ny = pos(cx, 1 - cy, cz)
        nz = pos(cx, cy, 1 - cz)
        fxy = pos(1 - cx, 1 - cy, cz)
        fxz = pos(1 - cx, cy, 1 - cz)
        fyz = pos(cx, 1 - cy, 1 - cz)
        fxyz = pos(1 - cx, 1 - cy, 1 - cz)

        xv = x_ref[:, :]
        scores = jnp.dot(xv, rw_ref[:, :], preferred_element_type=jnp.float32)
        s_max = jnp.max(scores, axis=-1, keepdims=True)
        p = jnp.exp(scores - s_max)
        probs = p / jnp.sum(p, axis=-1, keepdims=True)
        e0 = idx_ref[:, 0:1]
        e1 = idx_ref[:, 1:2]
        eids = lax.broadcasted_iota(jnp.int32, scores.shape, 1)
        g0 = jnp.sum(jnp.where(eids == e0, probs, 0.0), axis=-1, keepdims=True)
        g1 = jnp.sum(jnp.where(eids == e1, probs, 0.0), axis=-1, keepdims=True)
        gs = g0 + g1
        g0 = g0 / gs
        g1 = g1 / gs

        my_q[...] = ew_ref[...].astype(COMM_DTYPE)

        def accum(block_ref, owner, first=False):
            ws = []
            for t in range(E_PER):
                e_id = owner * E_PER + t
                w = jnp.where(e0 == e_id, g0, 0.0) + jnp.where(e1 == e_id, g1, 0.0)
                ws.append(xv * w)
            xcat = jnp.concatenate(ws, axis=1)
            wmat = block_ref[...].reshape(E_PER * d, h)
            if wmat.dtype != jnp.float32:
                xcat = xcat.astype(wmat.dtype)
            contrib = jnp.dot(xcat, wmat, preferred_element_type=jnp.float32)
            out_ref[:, :] = contrib if first else out_ref[:, :] + contrib

        def copy(src, dst, dev, row, link):
            return pltpu.make_async_remote_copy(
                src_ref=src, dst_ref=dst,
                send_sem=send_sems.at[row, link],
                recv_sem=recv_sems.at[row, link],
                device_id=(dev,), device_id_type=pltpu.DeviceIdType.MESH,
            )

        barrier_sem = pltpu.get_barrier_semaphore()
        for nbr in (nx, ny, nz):
            pl.semaphore_signal(
                barrier_sem, inc=1,
                device_id=(nbr,), device_id_type=pltpu.DeviceIdType.MESH,
            )
        pl.semaphore_wait(barrier_sem, 3)

        HALF = [slice(0, 2), slice(2, 4)]
        devs = (nx, ny, nz)

        p1 = [
            [copy(my_q.at[HALF[hh]], all_ref.at[my, HALF[hh]], devs[l], hh, l)
             for hh in range(2)]
            for l in range(3)
        ]
        for l in range(3):
            for hh in range(2):
                p1[l][hh].start()

        accum(ew_ref, my, first=True)

        src_slot = (ny, nz, nx)
        p2 = [
            [copy(all_ref.at[src_slot[l], HALF[hh]],
                  all_ref.at[src_slot[l], HALF[hh]], devs[l], 2 + hh, l)
             for hh in range(2)]
            for l in range(3)
        ]
        unlock = [(1, 0), (2, 1), (0, 2)]
        for hh in range(2):
            for rl, sl in unlock:
                p1[rl][hh].wait_recv()
                p2[sl][hh].start()

        accum(all_ref.at[nx], nx)
        accum(all_ref.at[ny], ny)
        accum(all_ref.at[nz], nz)

        hh0 = slice(0, h // 2)
        hh1 = slice(h // 2, h)
        p3 = [
            copy(all_ref.at[fyz, 0:1], all_ref.at[fyz, 0:1], nx, 4, 0),
            copy(all_ref.at[fyz, 1:2, :, hh0], all_ref.at[fyz, 1:2, :, hh0], nx, 5, 0),
            copy(all_ref.at[fxz, 1:2, :, hh1], all_ref.at[fxz, 1:2, :, hh1], ny, 4, 1),
            copy(all_ref.at[fxz, 2:3], all_ref.at[fxz, 2:3], ny, 5, 1),
            copy(all_ref.at[fxy, 3:4], all_ref.at[fxy, 3:4], nz, 4, 2),
        ]
        p2[1][0].wait_recv()
        p3[0].start()
        p3[1].start()
        p2[2][0].wait_recv()
        p3[2].start()
        p2[2][1].wait_recv()
        p3[3].start()
        p2[0][0].wait_recv()
        p2[0][1].wait_recv()
        p3[4].start()
        p2[1][1].wait_recv()

        accum(all_ref.at[fxy], fxy)
        accum(all_ref.at[fyz], fyz)
        accum(all_ref.at[fxz], fxz)

        for r in p3:
            r.wait_recv()
        accum(all_ref.at[fxyz], fxyz)

        for l in range(3):
            for hh in range(2):
                p1[l][hh].wait_send()
                p2[l][hh].wait_send()
        for r in p3:
            r.wait_send()

    return pl.pallas_call(
        body,
        out_shape=jax.ShapeDtypeStruct((n_tok, h), jnp.float32),
        in_specs=[pl.BlockSpec(memory_space=pltpu.VMEM)] * 4,
        out_specs=pl.BlockSpec(memory_space=pltpu.VMEM),
        scratch_shapes=[
            pltpu.VMEM((N_DEV, e_per, d, h), COMM_DTYPE),
            pltpu.VMEM((e_per, d, h), COMM_DTYPE),
            pltpu.SemaphoreType.DMA((6, 3)),
            pltpu.SemaphoreType.DMA((6, 3)),
        ],
        compiler_params=pltpu.CompilerParams(collective_id=0),
    )(x, router_W, route_idx, expert_W)


# device time: 39905 ns/iter; 4.6496x vs baseline; 1.0110x over previous
import jax
import jax.numpy as jnp
from jax import lax
from jax.experimental import pallas as pl
from jax.experimental.pallas import tpu as pltpu

N_DEV = 8
E_PER = 4
COMM_DTYPE = jnp.bfloat16


def kernel(x, router_W, route_idx, expert_W):
    n_tok, d = x.shape
    e_per, _, h = expert_W.shape

    def body(x_ref, rw_ref, idx_ref, ew_ref, out_ref, all_ref, my_q, xcats,
             send_sems, recv_sems):
        my = lax.axis_index("i")

        q = lax.rem(my, 4)
        cz = my // 4
        cy = q // 2
        cx = lax.rem(q + cy, 2)

        def pos(px, py, pz):
            return pz * 4 + 2 * py + lax.rem(px + py, 2)

        nx = pos(1 - cx, cy, cz)
        ny = pos(cx, 1 - cy, cz)
        nz = pos(cx, cy, 1 - cz)
        fxy = pos(1 - cx, 1 - cy, cz)
        fxz = pos(1 - cx, cy, 1 - cz)
        fyz = pos(cx, 1 - cy, 1 - cz)
        fxyz = pos(1 - cx, 1 - cy, 1 - cz)

        xv = x_ref[:, :]
        scores = jnp.dot(xv, rw_ref[:, :], preferred_element_type=jnp.float32)
        s_max = jnp.max(scores, axis=-1, keepdims=True)
        p = jnp.exp(scores - s_max)
        probs = p / jnp.sum(p, axis=-1, keepdims=True)
        e0 = idx_ref[:, 0:1]
        e1 = idx_ref[:, 1:2]
        eids = lax.broadcasted_iota(jnp.int32, scores.shape, 1)
        g0 = jnp.sum(jnp.where(eids == e0, probs, 0.0), axis=-1, keepdims=True)
        g1 = jnp.sum(jnp.where(eids == e1, probs, 0.0), axis=-1, keepdims=True)
        gs = g0 + g1
        g0 = g0 / gs
        g1 = g1 / gs

        my_q[...] = ew_ref[...].astype(COMM_DTYPE)

        def build_xcat(slot, owner):
            ws = []
            for t in range(E_PER):
                e_id = owner * E_PER + t
                w = jnp.where(e0 == e_id, g0, 0.0) + jnp.where(e1 == e_id, g1, 0.0)
                ws.append(xv * w)
            xcats[slot] = jnp.concatenate(ws, axis=1).astype(COMM_DTYPE)

        def accum(slot, block_ref, first=False):
            wmat = block_ref[...].astype(COMM_DTYPE).reshape(E_PER * d, h)
            contrib = jnp.dot(xcats[slot], wmat, preferred_element_type=jnp.float32)
            out_ref[:, :] = contrib if first else out_ref[:, :] + contrib

        def copy(src, dst, dev, row, link):
            return pltpu.make_async_remote_copy(
                src_ref=src, dst_ref=dst,
                send_sem=send_sems.at[row, link],
                recv_sem=recv_sems.at[row, link],
                device_id=(dev,), device_id_type=pltpu.DeviceIdType.MESH,
            )

        barrier_sem = pltpu.get_barrier_semaphore()
        for nbr in (nx, ny, nz):
            pl.semaphore_signal(
                barrier_sem, inc=1,
                device_id=(nbr,), device_id_type=pltpu.DeviceIdType.MESH,
            )
        pl.semaphore_wait(barrier_sem, 3)

        HALF = [slice(0, 2), slice(2, 4)]
        devs = (nx, ny, nz)

        p1 = [
            [copy(my_q.at[HALF[hh]], all_ref.at[my, HALF[hh]], devs[l], hh, l)
             for hh in range(2)]
            for l in range(3)
        ]
        for l in range(3):
            for hh in range(2):
                p1[l][hh].start()

        for slot, owner in enumerate((my, nx, ny, nz, fxy, fyz, fxz, fxyz)):
            build_xcat(slot, owner)
        accum(0, my_q, first=True)

        src_slot = (ny, nz, nx)
        p2 = [
            [copy(all_ref.at[src_slot[l], HALF[hh]],
                  all_ref.at[src_slot[l], HALF[hh]], devs[l], 2 + hh, l)
             for hh in range(2)]
            for l in range(3)
        ]
        unlock = [(1, 0), (2, 1), (0, 2)]
        for hh in range(2):
            for rl, sl in unlock:
                p1[rl][hh].wait_recv()
                p2[sl][hh].start()

        accum(1, all_ref.at[nx])
        accum(2, all_ref.at[ny])
        accum(3, all_ref.at[nz])

        hh0 = slice(0, h // 2)
        hh1 = slice(h // 2, h)
        p3 = [
            copy(all_ref.at[fyz, 0:1], all_ref.at[fyz, 0:1], nx, 4, 0),
            copy(all_ref.at[fyz, 1:2, :, hh0], all_ref.at[fyz, 1:2, :, hh0], nx, 5, 0),
            copy(all_ref.at[fxz, 1:2, :, hh1], all_ref.at[fxz, 1:2, :, hh1], ny, 4, 1),
            copy(all_ref.at[fxz, 2:3], all_ref.at[fxz, 2:3], ny, 5, 1),
            copy(all_ref.at[fxy, 3:4], all_ref.at[fxy, 3:4], nz, 4, 2),
        ]
        p2[1][0].wait_recv()
        p3[0].start()
        p3[1].start()
        p2[2][0].wait_recv()
        p3[2].start()
        p2[2][1].wait_recv()
        p3[3].start()
        p2[0][0].wait_recv()
        p2[0][1].wait_recv()
        p3[4].start()
        p2[1][1].wait_recv()

        accum(4, all_ref.at[fxy])
        accum(5, all_ref.at[fyz])
        accum(6, all_ref.at[fxz])

        for r in p3:
            r.wait_recv()
        accum(7, all_ref.at[fxyz])

        for l in range(3):
            for hh in range(2):
                p1[l][hh].wait_send()
                p2[l][hh].wait_send()
        for r in p3:
            r.wait_send()

    return pl.pallas_call(
        body,
        out_shape=jax.ShapeDtypeStruct((n_tok, h), jnp.float32),
        in_specs=[pl.BlockSpec(memory_space=pltpu.VMEM)] * 4,
        out_specs=pl.BlockSpec(memory_space=pltpu.VMEM),
        scratch_shapes=[
            pltpu.VMEM((N_DEV, e_per, d, h), COMM_DTYPE),
            pltpu.VMEM((e_per, d, h), COMM_DTYPE),
            pltpu.VMEM((N_DEV, n_tok, e_per * d), COMM_DTYPE),
            pltpu.SemaphoreType.DMA((6, 3)),
            pltpu.SemaphoreType.DMA((6, 3)),
        ],
        compiler_params=pltpu.CompilerParams(collective_id=0),
    )(x, router_W, route_idx, expert_W)
